# baseline (speedup 1.0000x reference)
import functools

import jax
import jax.numpy as jnp
from jax import lax
from jax.experimental import pallas as pl
from jax.experimental.pallas import tpu as pltpu
from jax.experimental.pallas import tpu_sc as plsc

_NUM_MEMORY = 100000
_NUM_FEATURES = 128
_BATCH = 1024
_TEMP = 0.05
_MOMENTUM = 0.2

_NSUB = 16
_ROWS_MAIN = 6256
_ROWS_LAST = _NUM_MEMORY - 15 * _ROWS_MAIN
_B_PER_SUB = _BATCH // _NSUB
_BN = 2048


def _mm_body(x_ref, f_ref, o_ref):
    acc = lax.dot_general(
        x_ref[...], f_ref[...], (((1,), (1,)), ((), ())),
        preferred_element_type=jnp.float32)
    o_ref[...] = acc / jnp.float32(_TEMP)


def _logits(inputs, features):
    grid = (pl.cdiv(_NUM_MEMORY, _BN),)
    return pl.pallas_call(
        _mm_body,
        grid=grid,
        in_specs=[
            pl.BlockSpec((_BATCH, _NUM_FEATURES), lambda i: (0, 0)),
            pl.BlockSpec((_BN, _NUM_FEATURES), lambda i: (i, 0)),
        ],
        out_specs=pl.BlockSpec((_BATCH, _BN), lambda i: (0, i)),
        out_shape=jax.ShapeDtypeStruct((_BATCH, _NUM_MEMORY), jnp.float32),
        compiler_params=pltpu.CompilerParams(
            dimension_semantics=("arbitrary",)),
    )(inputs, features)


def _update_body(x_hbm, idx_hbm, f_hbm, out_hbm,
                 idxall, myidx, wb, fsel, xsel,
                 cp_sem, g1_sem, g2_sem, sc_sem):
    s = lax.axis_index("s")
    base_rows = jnp.minimum(s * _ROWS_MAIN, _NUM_MEMORY - _ROWS_MAIN)
    base_b = s * _B_PER_SUB

    cp = pltpu.async_copy(
        f_hbm.at[pl.ds(base_rows, _ROWS_MAIN)],
        out_hbm.at[pl.ds(base_rows, _ROWS_MAIN)],
        cp_sem)

    pltpu.sync_copy(idx_hbm, idxall)
    pltpu.sync_copy(idx_hbm.at[pl.ds(base_b, _B_PER_SUB)], myidx)

    lane = lax.iota(jnp.int32, 16)
    _dnums = lax.GatherDimensionNumbers(
        offset_dims=(), collapsed_slice_dims=(0,), start_index_map=(0,))

    def _splat(vec, l):
        idxv = jnp.full((16, 1), l, jnp.int32)
        return lax.gather(vec, idxv, _dnums, slice_sizes=(1,),
                          mode=lax.GatherScatterMode.PROMISE_IN_BOUNDS)

    def scan_entry_chunk(jc, _):
        mychunk = myidx[pl.ds(jc * 16, 16)]

        def scan_lane(l, wbvec):
            tgt = _splat(mychunk, l)

            def scan_chunk(c, m):
                v = idxall[pl.ds(c * 16, 16)]
                cand = jnp.where(v == tgt, lane + c * 16, -1)
                return jnp.maximum(m, cand)

            m = lax.fori_loop(0, _BATCH // 16, scan_chunk,
                              jnp.full((16,), -1, jnp.int32))
            return jnp.where(lane == l, jnp.max(m), wbvec)

        wbvec = lax.fori_loop(0, 16, scan_lane,
                              jnp.zeros((16,), jnp.int32))
        wb[pl.ds(jc * 16, 16)] = wbvec
        return 0

    lax.fori_loop(0, _B_PER_SUB // 16, scan_entry_chunk, 0)

    pltpu.async_copy(f_hbm.at[myidx], fsel, g1_sem).wait()
    pltpu.async_copy(x_hbm.at[wb], xsel, g2_sem).wait()

    def update_row(j, _):
        def acc_chunk(c, a):
            u = (jnp.float32(_MOMENTUM) * fsel[j, pl.ds(c * 16, 16)]
                 + jnp.float32(1.0 - _MOMENTUM) * xsel[j, pl.ds(c * 16, 16)])
            fsel[j, pl.ds(c * 16, 16)] = u
            return a + u * u

        a = lax.fori_loop(0, _NUM_FEATURES // 16, acc_chunk,
                          jnp.zeros((16,), jnp.float32))
        sq = jnp.sum(a)
        ih = lax.bitcast_convert_type(sq, jnp.int32)
        y = lax.bitcast_convert_type(
            jnp.int32(0x5F3759DF) - lax.shift_right_arithmetic(ih, 1),
            jnp.float32)
        for _ in range(4):
            y = y * (jnp.float32(1.5) - jnp.float32(0.5) * sq * y * y)

        def scale_chunk(c, _):
            fsel[j, pl.ds(c * 16, 16)] = fsel[j, pl.ds(c * 16, 16)] * y
            return 0

        lax.fori_loop(0, _NUM_FEATURES // 16, scale_chunk, 0)
        return 0

    lax.fori_loop(0, _B_PER_SUB, update_row, 0)

    cp.wait()
    plsc.subcore_barrier()

    pltpu.async_copy(fsel, out_hbm.at[myidx], sc_sem).wait()


@jax.jit
def _update(inputs, indexes, features):
    mesh = plsc.VectorSubcoreMesh(
        core_axis_name="c", subcore_axis_name="s", num_cores=1)
    kfn = pl.kernel(
        _update_body,
        out_type=jax.ShapeDtypeStruct((_NUM_MEMORY, _NUM_FEATURES),
                                      jnp.float32),
        mesh=mesh,
        scratch_types=[
            pltpu.VMEM((_BATCH,), jnp.int32),
            pltpu.VMEM((_B_PER_SUB,), jnp.int32),
            pltpu.VMEM((_B_PER_SUB,), jnp.int32),
            pltpu.VMEM((_B_PER_SUB, _NUM_FEATURES), jnp.float32),
            pltpu.VMEM((_B_PER_SUB, _NUM_FEATURES), jnp.float32),
            pltpu.SemaphoreType.DMA,
            pltpu.SemaphoreType.DMA,
            pltpu.SemaphoreType.DMA,
            pltpu.SemaphoreType.DMA,
        ],
        compiler_params=pltpu.CompilerParams(needs_layout_passes=False),
    )
    return kfn(inputs, indexes, features)


def kernel(inputs, indexes, features):
    idx = indexes.astype(jnp.int32)
    outputs = _logits(inputs, features)
    new_features = _update(inputs, idx, features)
    return outputs, new_features


# speedup vs baseline: 2.8476x; 2.8476x over previous
import jax
import jax.numpy as jnp
from jax import lax
from jax.experimental import pallas as pl
from jax.experimental.pallas import tpu as pltpu
from jax.experimental.pallas import tpu_sc as plsc

_NUM_MEMORY = 100000
_NUM_FEATURES = 128
_BATCH = 1024
_TEMP = 0.05
_MOMENTUM = 0.2

_NSUB = 16
_ROWS_MAIN = 6256
_B_PER_SUB = _BATCH // _NSUB
_CH = 368
_NCH = 17
_BN = 2048


def _mm_body(x_ref, f_ref, o_ref):
    acc = lax.dot_general(
        x_ref[...], f_ref[...], (((1,), (1,)), ((), ())),
        preferred_element_type=jnp.float32)
    o_ref[...] = acc / jnp.float32(_TEMP)


def _logits(inputs, features):
    grid = (pl.cdiv(_NUM_MEMORY, _BN),)
    return pl.pallas_call(
        _mm_body,
        grid=grid,
        in_specs=[
            pl.BlockSpec((_BATCH, _NUM_FEATURES), lambda i: (0, 0)),
            pl.BlockSpec((_BN, _NUM_FEATURES), lambda i: (i, 0)),
        ],
        out_specs=pl.BlockSpec((_BATCH, _BN), lambda i: (0, i)),
        out_shape=jax.ShapeDtypeStruct((_BATCH, _NUM_MEMORY), jnp.float32),
        compiler_params=pltpu.CompilerParams(
            dimension_semantics=("arbitrary",)),
    )(inputs, features)


def _wb_body(idx_r_ref, idx_c_ref, wb_ref):
    idx_r = idx_r_ref[...]
    idx_c = idx_c_ref[...]
    jj = lax.broadcasted_iota(jnp.int32, (_BATCH, _BATCH), 1)
    cand = jnp.where(idx_r == idx_c, jj, -1)
    wb_ref[...] = jnp.max(cand, axis=1, keepdims=True)


def _last_occurrence(idx):
    wb2d = pl.pallas_call(
        _wb_body,
        out_shape=jax.ShapeDtypeStruct((_BATCH, 1), jnp.int32),
    )(idx.reshape(_BATCH, 1), idx.reshape(1, _BATCH))
    return wb2d.reshape(_BATCH)


def _update_body(x_hbm, idx_hbm, wb_hbm, f_hbm, out_hbm,
                 myidx, mywb, fsel, xsel, cb0, cb1,
                 g1_sem, g2_sem, sc_sem, si0, si1, so0, so1):
    s = lax.axis_index("s")
    base_rows = jnp.minimum(s * _ROWS_MAIN, _NUM_MEMORY - _ROWS_MAIN)
    base_b = s * _B_PER_SUB

    bufs = (cb0, cb1)
    sin = (si0, si1)
    sout = (so0, so1)

    h_in = {}
    h_out = {}
    h_in[0] = pltpu.async_copy(
        f_hbm.at[pl.ds(base_rows, _CH)], cb0, si0)
    h_in[1] = pltpu.async_copy(
        f_hbm.at[pl.ds(base_rows + _CH, _CH)], cb1, si1)

    pltpu.sync_copy(idx_hbm.at[pl.ds(base_b, _B_PER_SUB)], myidx)
    pltpu.sync_copy(wb_hbm.at[pl.ds(base_b, _B_PER_SUB)], mywb)

    hg1 = pltpu.async_copy(f_hbm.at[myidx], fsel, g1_sem)
    hg2 = pltpu.async_copy(x_hbm.at[mywb], xsel, g2_sem)
    hg1.wait()
    hg2.wait()

    def update_row(j, _):
        def acc_chunk(c, a):
            u = (jnp.float32(_MOMENTUM) * fsel[j, pl.ds(c * 16, 16)]
                 + jnp.float32(1.0 - _MOMENTUM) * xsel[j, pl.ds(c * 16, 16)])
            fsel[j, pl.ds(c * 16, 16)] = u
            return a + u * u

        a = lax.fori_loop(0, _NUM_FEATURES // 16, acc_chunk,
                          jnp.zeros((16,), jnp.float32))
        sq = jnp.sum(a)
        ih = lax.bitcast_convert_type(sq, jnp.int32)
        y = lax.bitcast_convert_type(
            jnp.int32(0x5F3759DF) - lax.shift_right_arithmetic(ih, 1),
            jnp.float32)
        for _ in range(4):
            y = y * (jnp.float32(1.5) - jnp.float32(0.5) * sq * y * y)

        def scale_chunk(c, _):
            fsel[j, pl.ds(c * 16, 16)] = fsel[j, pl.ds(c * 16, 16)] * y
            return 0

        lax.fori_loop(0, _NUM_FEATURES // 16, scale_chunk, 0)
        return 0

    lax.fori_loop(0, _B_PER_SUB, update_row, 0)

    for i in range(_NCH):
        b = i & 1
        if i >= 2:
            h_out[i - 2].wait()
            h_in[i] = pltpu.async_copy(
                f_hbm.at[pl.ds(base_rows + i * _CH, _CH)], bufs[b], sin[b])
        h_in[i].wait()
        h_out[i] = pltpu.async_copy(
            bufs[b], out_hbm.at[pl.ds(base_rows + i * _CH, _CH)], sout[b])
    h_out[_NCH - 2].wait()
    h_out[_NCH - 1].wait()

    plsc.subcore_barrier()

    pltpu.async_copy(fsel, out_hbm.at[myidx], sc_sem).wait()


@jax.jit
def _update(inputs, indexes, wb, features):
    mesh = plsc.VectorSubcoreMesh(
        core_axis_name="c", subcore_axis_name="s", num_cores=1)
    kfn = pl.kernel(
        _update_body,
        out_type=jax.ShapeDtypeStruct((_NUM_MEMORY, _NUM_FEATURES),
                                      jnp.float32),
        mesh=mesh,
        scratch_types=[
            pltpu.VMEM((_B_PER_SUB,), jnp.int32),
            pltpu.VMEM((_B_PER_SUB,), jnp.int32),
            pltpu.VMEM((_B_PER_SUB, _NUM_FEATURES), jnp.float32),
            pltpu.VMEM((_B_PER_SUB, _NUM_FEATURES), jnp.float32),
            pltpu.VMEM((_CH, _NUM_FEATURES), jnp.float32),
            pltpu.VMEM((_CH, _NUM_FEATURES), jnp.float32),
            pltpu.SemaphoreType.DMA,
            pltpu.SemaphoreType.DMA,
            pltpu.SemaphoreType.DMA,
            pltpu.SemaphoreType.DMA,
            pltpu.SemaphoreType.DMA,
            pltpu.SemaphoreType.DMA,
            pltpu.SemaphoreType.DMA,
        ],
        compiler_params=pltpu.CompilerParams(needs_layout_passes=False),
    )
    return kfn(inputs, indexes, wb, features)


def kernel(inputs, indexes, features):
    idx = indexes.astype(jnp.int32)
    wb = _last_occurrence(idx)
    outputs = _logits(inputs, features)
    new_features = _update(inputs, idx, wb, features)
    return outputs, new_features


# speedup vs baseline: 9.0631x; 3.1827x over previous
import jax
import jax.numpy as jnp
from jax import lax
from jax.experimental import pallas as pl
from jax.experimental.pallas import tpu as pltpu
from jax.experimental.pallas import tpu_sc as plsc

_NUM_MEMORY = 100000
_NUM_FEATURES = 128
_BATCH = 1024
_TEMP = 0.05
_MOMENTUM = 0.2

_NSUB = 16
_B_PER_SUB = _BATCH // _NSUB
_BN = 4096


def _mm_body(f_ref, x_ref, o_ref, fc_ref):
    x = x_ref[...] * (jnp.float32(1.0) / jnp.float32(_TEMP))
    o_ref[...] = lax.dot_general(
        f_ref[...], x, (((1,), (1,)), ((), ())),
        preferred_element_type=jnp.float32)
    fc_ref[...] = f_ref[...]


def _logits_t_and_copy(inputs, features):
    grid = (pl.cdiv(_NUM_MEMORY, _BN),)
    return pl.pallas_call(
        _mm_body,
        grid=grid,
        in_specs=[
            pl.BlockSpec((_BN, _NUM_FEATURES), lambda i: (i, 0)),
            pl.BlockSpec((_BATCH, _NUM_FEATURES), lambda i: (0, 0)),
        ],
        out_specs=[
            pl.BlockSpec((_BN, _BATCH), lambda i: (i, 0)),
            pl.BlockSpec((_BN, _NUM_FEATURES), lambda i: (i, 0)),
        ],
        out_shape=[
            jax.ShapeDtypeStruct((_NUM_MEMORY, _BATCH), jnp.float32),
            jax.ShapeDtypeStruct((_NUM_MEMORY, _NUM_FEATURES), jnp.float32),
        ],
        compiler_params=pltpu.CompilerParams(
            dimension_semantics=("parallel",)),
    )(features, inputs)


def _rows_body(x_hbm, idx_hbm, f_hbm, upd_hbm,
               idxall, myidx, mywb, fsel, xsel,
               g1_sem, g2_sem, st_sem):
    s = lax.axis_index("s")
    base_b = s * _B_PER_SUB

    pltpu.sync_copy(idx_hbm, idxall)
    pltpu.sync_copy(idx_hbm.at[pl.ds(base_b, _B_PER_SUB)], myidx)

    lane = lax.iota(jnp.int32, 16)
    _dnums = lax.GatherDimensionNumbers(
        offset_dims=(), collapsed_slice_dims=(0,), start_index_map=(0,))

    def _splat(vec, l):
        idxv = jnp.full((16, 1), l, jnp.int32)
        return lax.gather(vec, idxv, _dnums, slice_sizes=(1,),
                          mode=lax.GatherScatterMode.PROMISE_IN_BOUNDS)

    def scan_entry_chunk(jc, _):
        mychunk = myidx[pl.ds(jc * 16, 16)]

        def scan_lane(l, wbvec):
            tgt = _splat(mychunk, l)

            def scan_chunk(c, m):
                v = idxall[pl.ds(c * 16, 16)]
                cand = jnp.where(v == tgt, lane + c * 16, -1)
                return jnp.maximum(m, cand)

            m = lax.fori_loop(0, _BATCH // 16, scan_chunk,
                              jnp.full((16,), -1, jnp.int32))
            return jnp.where(lane == l, jnp.max(m), wbvec)

        wbvec = lax.fori_loop(0, 16, scan_lane, jnp.zeros((16,), jnp.int32))
        mywb[pl.ds(jc * 16, 16)] = wbvec
        return 0

    lax.fori_loop(0, _B_PER_SUB // 16, scan_entry_chunk, 0)

    hg1 = pltpu.async_copy(f_hbm.at[myidx], fsel, g1_sem)
    hg2 = pltpu.async_copy(x_hbm.at[mywb], xsel, g2_sem)
    hg1.wait()
    hg2.wait()

    def update_row(j, _):
        def acc_chunk(c, a):
            u = (jnp.float32(_MOMENTUM) * fsel[j, pl.ds(c * 16, 16)]
                 + jnp.float32(1.0 - _MOMENTUM) * xsel[j, pl.ds(c * 16, 16)])
            fsel[j, pl.ds(c * 16, 16)] = u
            return a + u * u

        a = lax.fori_loop(0, _NUM_FEATURES // 16, acc_chunk,
                          jnp.zeros((16,), jnp.float32))
        sq = jnp.sum(a)
        ih = lax.bitcast_convert_type(sq, jnp.int32)
        y = lax.bitcast_convert_type(
            jnp.int32(0x5F3759DF) - lax.shift_right_arithmetic(ih, 1),
            jnp.float32)
        for _ in range(4):
            y = y * (jnp.float32(1.5) - jnp.float32(0.5) * sq * y * y)

        def scale_chunk(c, _):
            fsel[j, pl.ds(c * 16, 16)] = fsel[j, pl.ds(c * 16, 16)] * y
            return 0

        lax.fori_loop(0, _NUM_FEATURES // 16, scale_chunk, 0)
        return 0

    lax.fori_loop(0, _B_PER_SUB, update_row, 0)

    pltpu.async_copy(
        fsel, upd_hbm.at[pl.ds(base_b, _B_PER_SUB)], st_sem).wait()


def _updated_rows(inputs, indexes, features):
    mesh = plsc.VectorSubcoreMesh(
        core_axis_name="c", subcore_axis_name="s", num_cores=1)
    kfn = pl.kernel(
        _rows_body,
        out_type=jax.ShapeDtypeStruct((_BATCH, _NUM_FEATURES), jnp.float32),
        mesh=mesh,
        scratch_types=[
            pltpu.VMEM((_BATCH,), jnp.int32),
            pltpu.VMEM((_B_PER_SUB,), jnp.int32),
            pltpu.VMEM((_B_PER_SUB,), jnp.int32),
            pltpu.VMEM((_B_PER_SUB, _NUM_FEATURES), jnp.float32),
            pltpu.VMEM((_B_PER_SUB, _NUM_FEATURES), jnp.float32),
            pltpu.SemaphoreType.DMA,
            pltpu.SemaphoreType.DMA,
            pltpu.SemaphoreType.DMA,
        ],
        compiler_params=pltpu.CompilerParams(needs_layout_passes=False),
    )
    return kfn(inputs, indexes, features)


def _scatter_body(idx_hbm, upd_hbm, f_ref, myidx, rows, sc_sem):
    s = lax.axis_index("s")
    base_b = s * _B_PER_SUB
    pltpu.sync_copy(idx_hbm.at[pl.ds(base_b, _B_PER_SUB)], myidx)
    pltpu.sync_copy(upd_hbm.at[pl.ds(base_b, _B_PER_SUB)], rows)
    pltpu.async_copy(rows, f_ref.at[myidx], sc_sem).wait()


def _scatter_into(indexes, upd, f_ref):
    mesh = plsc.VectorSubcoreMesh(
        core_axis_name="c", subcore_axis_name="s", num_cores=1)
    kfn = pl.kernel(
        _scatter_body,
        out_type=(),
        mesh=mesh,
        scratch_types=[
            pltpu.VMEM((_B_PER_SUB,), jnp.int32),
            pltpu.VMEM((_B_PER_SUB, _NUM_FEATURES), jnp.float32),
            pltpu.SemaphoreType.DMA,
        ],
        compiler_params=pltpu.CompilerParams(needs_layout_passes=False),
    )
    kfn(indexes, upd, f_ref)


def kernel(inputs, indexes, features):
    idx = indexes.astype(jnp.int32)
    logits_t, fcopy = _logits_t_and_copy(inputs, features)
    upd = _updated_rows(inputs, idx, features)
    fref = jax.new_ref(fcopy)
    _scatter_into(idx, upd, fref)
    return logits_t.T, fref[...]


# speedup vs baseline: 9.0770x; 1.0015x over previous
import jax
import jax.numpy as jnp
from jax import lax
from jax.experimental import pallas as pl
from jax.experimental.pallas import tpu as pltpu
from jax.experimental.pallas import tpu_sc as plsc

_NUM_MEMORY = 100000
_NUM_FEATURES = 128
_BATCH = 1024
_TEMP = 0.05
_MOMENTUM = 0.2

_NSUB = 16
_B_PER_SUB = _BATCH // _NSUB
_BN = 5120


def _mm_body(f_ref, x_ref, o_ref, fc_ref):
    x = x_ref[...] * (jnp.float32(1.0) / jnp.float32(_TEMP))
    o_ref[...] = lax.dot_general(
        f_ref[...], x, (((1,), (1,)), ((), ())),
        preferred_element_type=jnp.float32)
    fc_ref[...] = f_ref[...]


def _logits_t_and_copy(inputs, features):
    grid = (pl.cdiv(_NUM_MEMORY, _BN),)
    return pl.pallas_call(
        _mm_body,
        grid=grid,
        in_specs=[
            pl.BlockSpec((_BN, _NUM_FEATURES), lambda i: (i, 0)),
            pl.BlockSpec((_BATCH, _NUM_FEATURES), lambda i: (0, 0)),
        ],
        out_specs=[
            pl.BlockSpec((_BN, _BATCH), lambda i: (i, 0)),
            pl.BlockSpec((_BN, _NUM_FEATURES), lambda i: (i, 0)),
        ],
        out_shape=[
            jax.ShapeDtypeStruct((_NUM_MEMORY, _BATCH), jnp.float32),
            jax.ShapeDtypeStruct((_NUM_MEMORY, _NUM_FEATURES), jnp.float32),
        ],
        compiler_params=pltpu.CompilerParams(
            dimension_semantics=("parallel",)),
    )(features, inputs)


def _rows_body(x_hbm, idx_hbm, f_hbm, upd_hbm,
               idxall, myidx, mywb, fsel, xsel,
               g1_sem, g2_sem, st_sem):
    s = lax.axis_index("s")
    base_b = s * _B_PER_SUB

    pltpu.sync_copy(idx_hbm, idxall)
    pltpu.sync_copy(idx_hbm.at[pl.ds(base_b, _B_PER_SUB)], myidx)

    lane = lax.iota(jnp.int32, 16)
    _dnums = lax.GatherDimensionNumbers(
        offset_dims=(), collapsed_slice_dims=(0,), start_index_map=(0,))

    def _splat(vec, l):
        idxv = jnp.full((16, 1), l, jnp.int32)
        return lax.gather(vec, idxv, _dnums, slice_sizes=(1,),
                          mode=lax.GatherScatterMode.PROMISE_IN_BOUNDS)

    def scan_entry_chunk(jc, _):
        mychunk = myidx[pl.ds(jc * 16, 16)]

        def scan_lane(l, wbvec):
            tgt = _splat(mychunk, l)

            def scan_chunk(c, m):
                v = idxall[pl.ds(c * 16, 16)]
                cand = jnp.where(v == tgt, lane + c * 16, -1)
                return jnp.maximum(m, cand)

            m = lax.fori_loop(0, _BATCH // 16, scan_chunk,
                              jnp.full((16,), -1, jnp.int32))
            return jnp.where(lane == l, jnp.max(m), wbvec)

        wbvec = lax.fori_loop(0, 16, scan_lane, jnp.zeros((16,), jnp.int32))
        mywb[pl.ds(jc * 16, 16)] = wbvec
        return 0

    lax.fori_loop(0, _B_PER_SUB // 16, scan_entry_chunk, 0)

    hg1 = pltpu.async_copy(f_hbm.at[myidx], fsel, g1_sem)
    hg2 = pltpu.async_copy(x_hbm.at[mywb], xsel, g2_sem)
    hg1.wait()
    hg2.wait()

    def update_row(j, _):
        def acc_chunk(c, a):
            u = (jnp.float32(_MOMENTUM) * fsel[j, pl.ds(c * 16, 16)]
                 + jnp.float32(1.0 - _MOMENTUM) * xsel[j, pl.ds(c * 16, 16)])
            fsel[j, pl.ds(c * 16, 16)] = u
            return a + u * u

        a = lax.fori_loop(0, _NUM_FEATURES // 16, acc_chunk,
                          jnp.zeros((16,), jnp.float32))
        sq = jnp.sum(a)
        ih = lax.bitcast_convert_type(sq, jnp.int32)
        y = lax.bitcast_convert_type(
            jnp.int32(0x5F3759DF) - lax.shift_right_arithmetic(ih, 1),
            jnp.float32)
        for _ in range(4):
            y = y * (jnp.float32(1.5) - jnp.float32(0.5) * sq * y * y)

        def scale_chunk(c, _):
            fsel[j, pl.ds(c * 16, 16)] = fsel[j, pl.ds(c * 16, 16)] * y
            return 0

        lax.fori_loop(0, _NUM_FEATURES // 16, scale_chunk, 0)
        return 0

    lax.fori_loop(0, _B_PER_SUB, update_row, 0)

    pltpu.async_copy(
        fsel, upd_hbm.at[pl.ds(base_b, _B_PER_SUB)], st_sem).wait()


def _updated_rows(inputs, indexes, features):
    mesh = plsc.VectorSubcoreMesh(
        core_axis_name="c", subcore_axis_name="s", num_cores=1)
    kfn = pl.kernel(
        _rows_body,
        out_type=jax.ShapeDtypeStruct((_BATCH, _NUM_FEATURES), jnp.float32),
        mesh=mesh,
        scratch_types=[
            pltpu.VMEM((_BATCH,), jnp.int32),
            pltpu.VMEM((_B_PER_SUB,), jnp.int32),
            pltpu.VMEM((_B_PER_SUB,), jnp.int32),
            pltpu.VMEM((_B_PER_SUB, _NUM_FEATURES), jnp.float32),
            pltpu.VMEM((_B_PER_SUB, _NUM_FEATURES), jnp.float32),
            pltpu.SemaphoreType.DMA,
            pltpu.SemaphoreType.DMA,
            pltpu.SemaphoreType.DMA,
        ],
        compiler_params=pltpu.CompilerParams(needs_layout_passes=False),
    )
    return kfn(inputs, indexes, features)


def _scatter_body(idx_hbm, upd_hbm, f_ref, myidx, rows, sc_sem):
    s = lax.axis_index("s")
    base_b = s * _B_PER_SUB
    pltpu.sync_copy(idx_hbm.at[pl.ds(base_b, _B_PER_SUB)], myidx)
    pltpu.sync_copy(upd_hbm.at[pl.ds(base_b, _B_PER_SUB)], rows)
    pltpu.async_copy(rows, f_ref.at[myidx], sc_sem).wait()


def _scatter_into(indexes, upd, f_ref):
    mesh = plsc.VectorSubcoreMesh(
        core_axis_name="c", subcore_axis_name="s", num_cores=1)
    kfn = pl.kernel(
        _scatter_body,
        out_type=(),
        mesh=mesh,
        scratch_types=[
            pltpu.VMEM((_B_PER_SUB,), jnp.int32),
            pltpu.VMEM((_B_PER_SUB, _NUM_FEATURES), jnp.float32),
            pltpu.SemaphoreType.DMA,
        ],
        compiler_params=pltpu.CompilerParams(needs_layout_passes=False),
    )
    kfn(indexes, upd, f_ref)


def kernel(inputs, indexes, features):
    idx = indexes.astype(jnp.int32)
    logits_t, fcopy = _logits_t_and_copy(inputs, features)
    upd = _updated_rows(inputs, idx, features)
    fref = jax.new_ref(fcopy)
    _scatter_into(idx, upd, fref)
    return logits_t.T, fref[...]


# speedup vs baseline: 9.0786x; 1.0002x over previous
import jax
import jax.numpy as jnp
from jax import lax
from jax.experimental import pallas as pl
from jax.experimental.pallas import tpu as pltpu
from jax.experimental.pallas import tpu_sc as plsc

_NUM_MEMORY = 100000
_NUM_FEATURES = 128
_BATCH = 1024
_TEMP = 0.05
_MOMENTUM = 0.2

_NSUB = 16
_B_PER_SUB = _BATCH // _NSUB
_BN = 5120


def _mm_body(f_ref, x_ref, o_ref, fc_ref):
    acc = lax.dot_general(
        f_ref[...], x_ref[...], (((1,), (1,)), ((), ())),
        preferred_element_type=jnp.float32)
    o_ref[...] = acc / jnp.float32(_TEMP)
    fc_ref[...] = f_ref[...]


def _logits_t_and_copy(inputs, features):
    grid = (pl.cdiv(_NUM_MEMORY, _BN),)
    return pl.pallas_call(
        _mm_body,
        grid=grid,
        in_specs=[
            pl.BlockSpec((_BN, _NUM_FEATURES), lambda i: (i, 0)),
            pl.BlockSpec((_BATCH, _NUM_FEATURES), lambda i: (0, 0)),
        ],
        out_specs=[
            pl.BlockSpec((_BN, _BATCH), lambda i: (i, 0)),
            pl.BlockSpec((_BN, _NUM_FEATURES), lambda i: (i, 0)),
        ],
        out_shape=[
            jax.ShapeDtypeStruct((_NUM_MEMORY, _BATCH), jnp.float32),
            jax.ShapeDtypeStruct((_NUM_MEMORY, _NUM_FEATURES), jnp.float32),
        ],
        compiler_params=pltpu.CompilerParams(
            dimension_semantics=("parallel",)),
    )(features, inputs)


def _rows_body(x_hbm, idx_hbm, f_hbm, upd_hbm,
               idxall, myidx, mywb, fsel, xsel,
               g1_sem, g2_sem, st_sem):
    s = lax.axis_index("s")
    base_b = s * _B_PER_SUB

    pltpu.sync_copy(idx_hbm, idxall)
    pltpu.sync_copy(idx_hbm.at[pl.ds(base_b, _B_PER_SUB)], myidx)

    lane = lax.iota(jnp.int32, 16)
    _dnums = lax.GatherDimensionNumbers(
        offset_dims=(), collapsed_slice_dims=(0,), start_index_map=(0,))

    def _splat(vec, l):
        idxv = jnp.full((16, 1), l, jnp.int32)
        return lax.gather(vec, idxv, _dnums, slice_sizes=(1,),
                          mode=lax.GatherScatterMode.PROMISE_IN_BOUNDS)

    def scan_entry_chunk(jc, _):
        mychunk = myidx[pl.ds(jc * 16, 16)]

        def scan_lane(l, wbvec):
            tgt = _splat(mychunk, l)

            def scan_chunk(c, m):
                v = idxall[pl.ds(c * 16, 16)]
                cand = jnp.where(v == tgt, lane + c * 16, -1)
                return jnp.maximum(m, cand)

            m = lax.fori_loop(0, _BATCH // 16, scan_chunk,
                              jnp.full((16,), -1, jnp.int32))
            return jnp.where(lane == l, jnp.max(m), wbvec)

        wbvec = lax.fori_loop(0, 16, scan_lane, jnp.zeros((16,), jnp.int32))
        mywb[pl.ds(jc * 16, 16)] = wbvec
        return 0

    lax.fori_loop(0, _B_PER_SUB // 16, scan_entry_chunk, 0)

    hg1 = pltpu.async_copy(f_hbm.at[myidx], fsel, g1_sem)
    hg2 = pltpu.async_copy(x_hbm.at[mywb], xsel, g2_sem)
    hg1.wait()
    hg2.wait()

    def update_row(j, _):
        def acc_chunk(c, a):
            u = (jnp.float32(_MOMENTUM) * fsel[j, pl.ds(c * 16, 16)]
                 + jnp.float32(1.0 - _MOMENTUM) * xsel[j, pl.ds(c * 16, 16)])
            fsel[j, pl.ds(c * 16, 16)] = u
            return a + u * u

        a = lax.fori_loop(0, _NUM_FEATURES // 16, acc_chunk,
                          jnp.zeros((16,), jnp.float32))
        sq = jnp.sum(a)
        ih = lax.bitcast_convert_type(sq, jnp.int32)
        y = lax.bitcast_convert_type(
            jnp.int32(0x5F3759DF) - lax.shift_right_arithmetic(ih, 1),
            jnp.float32)
        for _ in range(4):
            y = y * (jnp.float32(1.5) - jnp.float32(0.5) * sq * y * y)

        def scale_chunk(c, _):
            fsel[j, pl.ds(c * 16, 16)] = fsel[j, pl.ds(c * 16, 16)] * y
            return 0

        lax.fori_loop(0, _NUM_FEATURES // 16, scale_chunk, 0)
        return 0

    lax.fori_loop(0, _B_PER_SUB, update_row, 0)

    pltpu.async_copy(
        fsel, upd_hbm.at[pl.ds(base_b, _B_PER_SUB)], st_sem).wait()


def _updated_rows(inputs, indexes, features):
    mesh = plsc.VectorSubcoreMesh(
        core_axis_name="c", subcore_axis_name="s", num_cores=1)
    kfn = pl.kernel(
        _rows_body,
        out_type=jax.ShapeDtypeStruct((_BATCH, _NUM_FEATURES), jnp.float32),
        mesh=mesh,
        scratch_types=[
            pltpu.VMEM((_BATCH,), jnp.int32),
            pltpu.VMEM((_B_PER_SUB,), jnp.int32),
            pltpu.VMEM((_B_PER_SUB,), jnp.int32),
            pltpu.VMEM((_B_PER_SUB, _NUM_FEATURES), jnp.float32),
            pltpu.VMEM((_B_PER_SUB, _NUM_FEATURES), jnp.float32),
            pltpu.SemaphoreType.DMA,
            pltpu.SemaphoreType.DMA,
            pltpu.SemaphoreType.DMA,
        ],
        compiler_params=pltpu.CompilerParams(needs_layout_passes=False),
    )
    return kfn(inputs, indexes, features)


def _scatter_body(idx_hbm, upd_hbm, f_ref, myidx, rows, sc_sem):
    s = lax.axis_index("s")
    base_b = s * _B_PER_SUB
    pltpu.sync_copy(idx_hbm.at[pl.ds(base_b, _B_PER_SUB)], myidx)
    pltpu.sync_copy(upd_hbm.at[pl.ds(base_b, _B_PER_SUB)], rows)
    pltpu.async_copy(rows, f_ref.at[myidx], sc_sem).wait()


def _scatter_into(indexes, upd, f_ref):
    mesh = plsc.VectorSubcoreMesh(
        core_axis_name="c", subcore_axis_name="s", num_cores=1)
    kfn = pl.kernel(
        _scatter_body,
        out_type=(),
        mesh=mesh,
        scratch_types=[
            pltpu.VMEM((_B_PER_SUB,), jnp.int32),
            pltpu.VMEM((_B_PER_SUB, _NUM_FEATURES), jnp.float32),
            pltpu.SemaphoreType.DMA,
        ],
        compiler_params=pltpu.CompilerParams(needs_layout_passes=False),
    )
    kfn(indexes, upd, f_ref)


def kernel(inputs, indexes, features):
    idx = indexes.astype(jnp.int32)
    logits_t, fcopy = _logits_t_and_copy(inputs, features)
    upd = _updated_rows(inputs, idx, features)
    fref = jax.new_ref(fcopy)
    _scatter_into(idx, upd, fref)
    return logits_t.T, fref[...]
